# baseline (speedup 1.0000x reference)
import functools

import jax
import jax.numpy as jnp
from jax import lax
from jax.experimental import pallas as pl
from jax.experimental.pallas import tpu as pltpu
from jax.experimental.pallas import tpu_sc as plsc

NC = 2
NS = 16
LANES = 16

CH = 128
RC = 13
PC = 3


def _stage_offsets(ps_hbm, ps_v, ps_sm, B, T):
    nmem = min(LANES, B)
    pltpu.sync_copy(ps_hbm.at[pl.ds(0, nmem)], ps_v.at[pl.ds(0, nmem)])
    for k0 in range(0, B, LANES):
        v = ps_v[pl.ds(k0, LANES)]
        for k in range(min(LANES, B - k0)):
            ps_sm[k0 + k] = v[k]
    ps_sm[B] = jnp.int32(T)


def _ln(z):
    bits = lax.bitcast_convert_type(z, jnp.int32)
    eb = bits >> 23
    m = lax.bitcast_convert_type(bits - (eb << 23) + (127 << 23), jnp.float32)
    e = eb - 127
    big = m > 1.4142135623730951
    m = jnp.where(big, m * 0.5, m)
    e = e + jnp.where(big, 1, 0)
    s = (m - 1.0) / (m + 1.0)
    s2 = s * s
    p = 1.0 + s2 * (1 / 3 + s2 * (1 / 5 + s2 * (1 / 7 + s2 * (1 / 9))))
    return e.astype(jnp.float32) * 0.6931471805599453 + 2.0 * s * p


def _make_fused(T, D, B):
    COLS = D // NC
    NJ = COLS // LANES
    RW = T // NS
    assert RW == (RC + PC) * CH
    assert B == NS
    SLAB = RC * CH
    mesh = plsc.VectorSubcoreMesh(core_axis_name="c", subcore_axis_name="s")

    @functools.partial(
        pl.kernel,
        out_type=(
            jax.ShapeDtypeStruct((T, D), jnp.float32),
            jax.ShapeDtypeStruct((NC, NS, B, COLS), jnp.float32),
            jax.ShapeDtypeStruct((NC, NS, B, COLS), jnp.float32),
        ),
        compiler_params=pltpu.CompilerParams(use_tc_tiling_on_sc=False),
        mesh=mesh,
        scratch_types=[
            pltpu.VMEM((SLAB, COLS), jnp.float32),
            pltpu.VMEM((CH, COLS), jnp.float32),
            pltpu.VMEM((CH, COLS), jnp.float32),
            pltpu.VMEM((B, COLS), jnp.float32),
            pltpu.VMEM((B, COLS), jnp.float32),
            pltpu.VMEM((B, COLS), jnp.float32),
            pltpu.VMEM((1, COLS), jnp.float32),
            pltpu.VMEM((LANES,), jnp.int32),
            pltpu.SMEM((32,), jnp.int32),
            pltpu.VMEM_SHARED((B, COLS), jnp.float32),
            pltpu.SemaphoreType.DMA,
            pltpu.SemaphoreType.DMA,
            pltpu.SemaphoreType.DMA,
            pltpu.SemaphoreType.DMA,
            pltpu.SemaphoreType.DMA,
            pltpu.SemaphoreType.DMA,
        ],
    )
    def fused(x_hbm, ps_hbm, out_hbm, pm_hbm, psm_hbm, slab, bnca, bncb,
              m_v, s_v, b_v, bseg_v, ps_v, ps_sm, b_sh,
              ssem, isem_a, isem_b, oslab, osem_a, osem_b):
        cid = lax.axis_index("c")
        sid = lax.axis_index("s")
        c0 = cid * COLS
        r_lo = sid * RW

        _stage_offsets(ps_hbm, ps_v, ps_sm, B, T)

        for ci in range(RC):
            pltpu.async_copy(
                x_hbm.at[pl.ds(r_lo + ci * CH, CH), pl.ds(c0, COLS)],
                slab.at[pl.ds(ci * CH, CH), :], ssem)
        bncs = (bnca, bncb)
        isems = (isem_a, isem_b)
        osems = (osem_a, osem_b)
        for cj in range(min(PC, 2)):
            pltpu.async_copy(
                x_hbm.at[pl.ds(r_lo + (RC + cj) * CH, CH), pl.ds(c0, COLS)],
                bncs[cj], isems[cj])

        neg = jnp.full((LANES,), -jnp.inf, jnp.float32)
        zero = jnp.zeros((LANES,), jnp.float32)

        def init_body(sb, carry):
            for j in range(NJ):
                m_v[sb, pl.ds(j * LANES, LANES)] = neg
                s_v[sb, pl.ds(j * LANES, LANES)] = zero
            return carry

        lax.fori_loop(0, B, init_body, 0)

        def accum(buf, base, lo_r, hi_r):
            def seg_body(sb, carry):
                r0 = jnp.clip(ps_sm[sb] - base, lo_r, hi_r)
                r1 = jnp.clip(ps_sm[sb + 1] - base, lo_r, hi_r)

                @pl.when(r1 > r0)
                def _():
                    @plsc.parallel_loop(r0, r1, unroll=8, carry=(neg,) * NJ)
                    def cmax(r, acc):
                        return tuple(
                            jnp.maximum(acc[j], buf[r, pl.ds(j * LANES, LANES)])
                            for j in range(NJ)
                        )

                    mnew = []
                    for j in range(NJ):
                        sl = pl.ds(j * LANES, LANES)
                        mo = m_v[sb, sl]
                        mn = jnp.maximum(mo, cmax[j])
                        s_v[sb, sl] = s_v[sb, sl] * jnp.exp(mo - mn)
                        m_v[sb, sl] = mn
                        mnew.append(mn)

                    @plsc.parallel_loop(r0, r1, unroll=8, carry=(zero,) * NJ)
                    def ssum(r, acc):
                        return tuple(
                            acc[j]
                            + jnp.exp(buf[r, pl.ds(j * LANES, LANES)] - mnew[j])
                            for j in range(NJ)
                        )

                    for j in range(NJ):
                        sl = pl.ds(j * LANES, LANES)
                        s_v[sb, sl] = s_v[sb, sl] + ssum[j]

                return carry

            lax.fori_loop(0, B, seg_body, 0)

        def res_body(ci, carry):
            pltpu.make_async_copy(
                x_hbm.at[pl.ds(r_lo + ci * CH, CH), pl.ds(c0, COLS)],
                slab.at[pl.ds(ci * CH, CH), :], ssem).wait()
            accum(slab, r_lo, ci * CH, ci * CH + CH)
            return carry

        lax.fori_loop(0, RC, res_body, 0)

        for cj in range(PC):
            base = r_lo + (RC + cj) * CH
            pltpu.make_async_copy(
                x_hbm.at[pl.ds(base, CH), pl.ds(c0, COLS)],
                bncs[cj % 2], isems[cj % 2]).wait()
            accum(bncs[cj % 2], base, 0, CH)
            if cj + 2 < PC:
                nbase = r_lo + (RC + cj + 2) * CH
                pltpu.async_copy(
                    x_hbm.at[pl.ds(nbase, CH), pl.ds(c0, COLS)],
                    bncs[cj % 2], isems[cj % 2])

        pltpu.sync_copy(m_v, pm_hbm.at[cid, sid])
        pltpu.sync_copy(s_v, psm_hbm.at[cid, sid])
        plsc.subcore_barrier()
        pltpu.sync_copy(pm_hbm.at[cid, :, sid, :], bnca.at[pl.ds(0, NS), :])
        pltpu.sync_copy(psm_hbm.at[cid, :, sid, :], bncb.at[pl.ds(0, NS), :])

        def mx_body(w, acc):
            return tuple(
                jnp.maximum(acc[j], bnca[w, pl.ds(j * LANES, LANES)])
                for j in range(NJ)
            )

        mseg = lax.fori_loop(0, NS, mx_body, (neg,) * NJ)

        def z_body(w, acc):
            out = []
            for j in range(NJ):
                sl = pl.ds(j * LANES, LANES)
                sp = bncb[w, sl]
                out.append(
                    acc[j]
                    + jnp.where(sp > 0, sp * jnp.exp(bnca[w, sl] - mseg[j]),
                                zero)
                )
            return tuple(out)

        zseg = lax.fori_loop(0, NS, z_body, (zero,) * NJ)

        for j in range(NJ):
            bseg_v[0, pl.ds(j * LANES, LANES)] = mseg[j] + _ln(zseg[j])
        pltpu.sync_copy(bseg_v, b_sh.at[pl.ds(sid, 1), :])
        plsc.subcore_barrier()
        pltpu.sync_copy(b_sh, b_v)

        for cj in range(min(PC, 2)):
            base = r_lo + (RC + cj) * CH
            pltpu.async_copy(
                x_hbm.at[pl.ds(base, CH), pl.ds(c0, COLS)],
                bncs[cj % 2], isems[cj % 2])

        def subtract(buf, base, lo_r, hi_r):
            def seg_body(sb, carry):
                r0 = jnp.clip(ps_sm[sb] - base, lo_r, hi_r)
                r1 = jnp.clip(ps_sm[sb + 1] - base, lo_r, hi_r)

                @pl.when(r1 > r0)
                def _():
                    bj = [b_v[sb, pl.ds(j * LANES, LANES)] for j in range(NJ)]

                    @plsc.parallel_loop(r0, r1, unroll=8)
                    def _sub(r):
                        for j in range(NJ):
                            sl = pl.ds(j * LANES, LANES)
                            buf[r, sl] = buf[r, sl] - bj[j]

                return carry

            lax.fori_loop(0, B, seg_body, 0)

        def out_body(ci, carry):
            subtract(slab, r_lo, ci * CH, ci * CH + CH)
            pltpu.async_copy(
                slab.at[pl.ds(ci * CH, CH), :],
                out_hbm.at[pl.ds(r_lo + ci * CH, CH), pl.ds(c0, COLS)], oslab)
            return carry

        lax.fori_loop(0, RC, out_body, 0)

        for cj in range(PC):
            base = r_lo + (RC + cj) * CH
            pltpu.make_async_copy(
                x_hbm.at[pl.ds(base, CH), pl.ds(c0, COLS)],
                bncs[cj % 2], isems[cj % 2]).wait()
            subtract(bncs[cj % 2], base, 0, CH)
            pltpu.async_copy(
                bncs[cj % 2],
                out_hbm.at[pl.ds(base, CH), pl.ds(c0, COLS)], osems[cj % 2])
            if cj + 2 < PC:
                pltpu.make_async_copy(
                    bncs[cj % 2],
                    out_hbm.at[pl.ds(base, CH), pl.ds(c0, COLS)],
                    osems[cj % 2]).wait()
                pltpu.async_copy(
                    x_hbm.at[pl.ds(base + 2 * CH, CH), pl.ds(c0, COLS)],
                    bncs[cj % 2], isems[cj % 2])

        def drain_body(ci, carry):
            pltpu.make_async_copy(
                slab.at[pl.ds(ci * CH, CH), :],
                out_hbm.at[pl.ds(r_lo + ci * CH, CH), pl.ds(c0, COLS)],
                oslab).wait()
            return carry

        lax.fori_loop(0, RC, drain_body, 0)
        for cj in range(max(PC - 2, 0), PC):
            base = r_lo + (RC + cj) * CH
            pltpu.make_async_copy(
                bncs[cj % 2],
                out_hbm.at[pl.ds(base, CH), pl.ds(c0, COLS)],
                osems[cj % 2]).wait()

    return fused


def kernel(logits, prefix_sum):
    T, D = logits.shape
    B = prefix_sum.shape[0] - 1
    out, _, _ = _make_fused(T, D, B)(logits, prefix_sum)
    return out


# speedup vs baseline: 1.0595x; 1.0595x over previous
import functools

import jax
import jax.numpy as jnp
from jax import lax
from jax.experimental import pallas as pl
from jax.experimental.pallas import tpu as pltpu
from jax.experimental.pallas import tpu_sc as plsc

NC = 2
NS = 16
LANES = 16

CH = 128
RC = 13
PC = 3


def _stage_offsets(ps_hbm, ps_v, ps_sm, B, T):
    nmem = min(LANES, B)
    pltpu.sync_copy(ps_hbm.at[pl.ds(0, nmem)], ps_v.at[pl.ds(0, nmem)])
    for k0 in range(0, B, LANES):
        v = ps_v[pl.ds(k0, LANES)]
        for k in range(min(LANES, B - k0)):
            ps_sm[k0 + k] = v[k]
    ps_sm[B] = jnp.int32(T)


def _ln(z):
    bits = lax.bitcast_convert_type(z, jnp.int32)
    eb = bits >> 23
    m = lax.bitcast_convert_type(bits - (eb << 23) + (127 << 23), jnp.float32)
    e = eb - 127
    big = m > 1.4142135623730951
    m = jnp.where(big, m * 0.5, m)
    e = e + jnp.where(big, 1, 0)
    s = (m - 1.0) / (m + 1.0)
    s2 = s * s
    p = 1.0 + s2 * (1 / 3 + s2 * (1 / 5 + s2 * (1 / 7 + s2 * (1 / 9))))
    return e.astype(jnp.float32) * 0.6931471805599453 + 2.0 * s * p


def _make_fused(T, D, B):
    COLS = D // NC
    NJ = COLS // LANES
    RW = T // NS
    assert RW == (RC + PC) * CH
    assert B == NS
    SLAB = RC * CH
    mesh = plsc.VectorSubcoreMesh(core_axis_name="c", subcore_axis_name="s")

    @functools.partial(
        pl.kernel,
        out_type=(
            jax.ShapeDtypeStruct((T, D), jnp.float32),
            jax.ShapeDtypeStruct((NC, NS, B, COLS), jnp.float32),
            jax.ShapeDtypeStruct((NC, NS, B, COLS), jnp.float32),
        ),
        compiler_params=pltpu.CompilerParams(use_tc_tiling_on_sc=False),
        mesh=mesh,
        scratch_types=[
            pltpu.VMEM((SLAB, COLS), jnp.float32),
            pltpu.VMEM((CH, COLS), jnp.float32),
            pltpu.VMEM((CH, COLS), jnp.float32),
            pltpu.VMEM((B, COLS), jnp.float32),
            pltpu.VMEM((B, COLS), jnp.float32),
            pltpu.VMEM((B, COLS), jnp.float32),
            pltpu.VMEM((1, COLS), jnp.float32),
            pltpu.VMEM((LANES,), jnp.int32),
            pltpu.SMEM((32,), jnp.int32),
            pltpu.VMEM_SHARED((B, COLS), jnp.float32),
            pltpu.SemaphoreType.DMA,
            pltpu.SemaphoreType.DMA,
            pltpu.SemaphoreType.DMA,
            pltpu.SemaphoreType.DMA,
            pltpu.SemaphoreType.DMA,
            pltpu.SemaphoreType.DMA,
        ],
    )
    def fused(x_hbm, ps_hbm, out_hbm, pm_hbm, psm_hbm, slab, bnca, bncb,
              m_v, s_v, b_v, bseg_v, ps_v, ps_sm, b_sh,
              ssem, isem_a, isem_b, oslab, osem_a, osem_b):
        cid = lax.axis_index("c")
        sid = lax.axis_index("s")
        c0 = cid * COLS
        r_lo = sid * RW

        _stage_offsets(ps_hbm, ps_v, ps_sm, B, T)

        for ci in range(RC):
            pltpu.async_copy(
                x_hbm.at[pl.ds(r_lo + ci * CH, CH), pl.ds(c0, COLS)],
                slab.at[pl.ds(ci * CH, CH), :], ssem)
        bncs = (bnca, bncb)
        isems = (isem_a, isem_b)
        osems = (osem_a, osem_b)
        for cj in range(min(PC, 2)):
            pltpu.async_copy(
                x_hbm.at[pl.ds(r_lo + (RC + cj) * CH, CH), pl.ds(c0, COLS)],
                bncs[cj], isems[cj])

        neg = jnp.full((LANES,), -jnp.inf, jnp.float32)
        zero = jnp.zeros((LANES,), jnp.float32)

        def init_body(sb, carry):
            for j in range(NJ):
                m_v[sb, pl.ds(j * LANES, LANES)] = neg
                s_v[sb, pl.ds(j * LANES, LANES)] = zero
            return carry

        lax.fori_loop(0, B, init_body, 0)

        def accum(buf, base, lo_r, hi_r, sb0):
            start = base + lo_r
            end = base + hi_r
            sb1 = jnp.int32(0)
            se = jnp.int32(0)
            for k in range(B):
                sb1 = sb1 + (ps_sm[k + 1] <= start).astype(jnp.int32)
                se = se + (ps_sm[k] < end).astype(jnp.int32)

            def seg_body(sb, carry):
                r0 = jnp.clip(ps_sm[sb] - base, lo_r, hi_r)
                r1 = jnp.clip(ps_sm[sb + 1] - base, lo_r, hi_r)

                @plsc.parallel_loop(r0, r1, unroll=8, carry=(neg,) * NJ)
                def cmax(r, acc):
                    return tuple(
                        jnp.maximum(acc[j], buf[r, pl.ds(j * LANES, LANES)])
                        for j in range(NJ)
                    )

                mnew = []
                for j in range(NJ):
                    sl = pl.ds(j * LANES, LANES)
                    mo = m_v[sb, sl]
                    mn = jnp.maximum(mo, cmax[j])
                    s_v[sb, sl] = s_v[sb, sl] * jnp.exp(mo - mn)
                    m_v[sb, sl] = mn
                    mnew.append(mn)

                @plsc.parallel_loop(r0, r1, unroll=8, carry=(zero,) * NJ)
                def ssum(r, acc):
                    return tuple(
                        acc[j]
                        + jnp.exp(buf[r, pl.ds(j * LANES, LANES)] - mnew[j])
                        for j in range(NJ)
                    )

                for j in range(NJ):
                    sl = pl.ds(j * LANES, LANES)
                    s_v[sb, sl] = s_v[sb, sl] + ssum[j]

                return carry

            lax.fori_loop(sb1, se, seg_body, 0)
            return se - 1

        sb_t = jnp.int32(0)

        def res_body(ci, sb):
            pltpu.make_async_copy(
                x_hbm.at[pl.ds(r_lo + ci * CH, CH), pl.ds(c0, COLS)],
                slab.at[pl.ds(ci * CH, CH), :], ssem).wait()
            return accum(slab, r_lo, ci * CH, ci * CH + CH, sb)

        sb_r = lax.fori_loop(0, RC, res_body, sb_t)

        for cj in range(PC):
            base = r_lo + (RC + cj) * CH
            pltpu.make_async_copy(
                x_hbm.at[pl.ds(base, CH), pl.ds(c0, COLS)],
                bncs[cj % 2], isems[cj % 2]).wait()
            sb_r = accum(bncs[cj % 2], base, 0, CH, sb_r)
            if cj + 2 < PC:
                nbase = r_lo + (RC + cj + 2) * CH
                pltpu.async_copy(
                    x_hbm.at[pl.ds(nbase, CH), pl.ds(c0, COLS)],
                    bncs[cj % 2], isems[cj % 2])

        pltpu.sync_copy(m_v, pm_hbm.at[cid, sid])
        pltpu.sync_copy(s_v, psm_hbm.at[cid, sid])
        plsc.subcore_barrier()
        pltpu.sync_copy(pm_hbm.at[cid, :, sid, :], bnca.at[pl.ds(0, NS), :])
        pltpu.sync_copy(psm_hbm.at[cid, :, sid, :], bncb.at[pl.ds(0, NS), :])

        def mx_body(w, acc):
            return tuple(
                jnp.maximum(acc[j], bnca[w, pl.ds(j * LANES, LANES)])
                for j in range(NJ)
            )

        mseg = lax.fori_loop(0, NS, mx_body, (neg,) * NJ)

        def z_body(w, acc):
            out = []
            for j in range(NJ):
                sl = pl.ds(j * LANES, LANES)
                sp = bncb[w, sl]
                out.append(
                    acc[j]
                    + jnp.where(sp > 0, sp * jnp.exp(bnca[w, sl] - mseg[j]),
                                zero)
                )
            return tuple(out)

        zseg = lax.fori_loop(0, NS, z_body, (zero,) * NJ)

        for j in range(NJ):
            bseg_v[0, pl.ds(j * LANES, LANES)] = mseg[j] + _ln(zseg[j])
        pltpu.sync_copy(bseg_v, b_sh.at[pl.ds(sid, 1), :])
        plsc.subcore_barrier()
        pltpu.sync_copy(b_sh, b_v)

        for cj in range(min(PC, 2)):
            base = r_lo + (RC + cj) * CH
            pltpu.async_copy(
                x_hbm.at[pl.ds(base, CH), pl.ds(c0, COLS)],
                bncs[cj % 2], isems[cj % 2])

        def subtract(buf, base, lo_r, hi_r, sb0):
            start = base + lo_r
            end = base + hi_r
            sb1 = jnp.int32(0)
            se = jnp.int32(0)
            for k in range(B):
                sb1 = sb1 + (ps_sm[k + 1] <= start).astype(jnp.int32)
                se = se + (ps_sm[k] < end).astype(jnp.int32)

            def seg_body(sb, carry):
                r0 = jnp.clip(ps_sm[sb] - base, lo_r, hi_r)
                r1 = jnp.clip(ps_sm[sb + 1] - base, lo_r, hi_r)
                bj = [b_v[sb, pl.ds(j * LANES, LANES)] for j in range(NJ)]

                @plsc.parallel_loop(r0, r1, unroll=8)
                def _sub(r):
                    for j in range(NJ):
                        sl = pl.ds(j * LANES, LANES)
                        buf[r, sl] = buf[r, sl] - bj[j]

                return carry

            lax.fori_loop(sb1, se, seg_body, 0)
            return se - 1

        def out_body(ci, sb):
            sb = subtract(slab, r_lo, ci * CH, ci * CH + CH, sb)
            pltpu.async_copy(
                slab.at[pl.ds(ci * CH, CH), :],
                out_hbm.at[pl.ds(r_lo + ci * CH, CH), pl.ds(c0, COLS)], oslab)
            return sb

        sb_c = lax.fori_loop(0, RC, out_body, sb_t)

        for cj in range(PC):
            base = r_lo + (RC + cj) * CH
            pltpu.make_async_copy(
                x_hbm.at[pl.ds(base, CH), pl.ds(c0, COLS)],
                bncs[cj % 2], isems[cj % 2]).wait()
            sb_c = subtract(bncs[cj % 2], base, 0, CH, sb_c)
            pltpu.async_copy(
                bncs[cj % 2],
                out_hbm.at[pl.ds(base, CH), pl.ds(c0, COLS)], osems[cj % 2])
            if cj + 2 < PC:
                pltpu.make_async_copy(
                    bncs[cj % 2],
                    out_hbm.at[pl.ds(base, CH), pl.ds(c0, COLS)],
                    osems[cj % 2]).wait()
                pltpu.async_copy(
                    x_hbm.at[pl.ds(base + 2 * CH, CH), pl.ds(c0, COLS)],
                    bncs[cj % 2], isems[cj % 2])

        def drain_body(ci, carry):
            pltpu.make_async_copy(
                slab.at[pl.ds(ci * CH, CH), :],
                out_hbm.at[pl.ds(r_lo + ci * CH, CH), pl.ds(c0, COLS)],
                oslab).wait()
            return carry

        lax.fori_loop(0, RC, drain_body, 0)
        for cj in range(max(PC - 2, 0), PC):
            base = r_lo + (RC + cj) * CH
            pltpu.make_async_copy(
                bncs[cj % 2],
                out_hbm.at[pl.ds(base, CH), pl.ds(c0, COLS)],
                osems[cj % 2]).wait()

    return fused


def kernel(logits, prefix_sum):
    T, D = logits.shape
    B = prefix_sum.shape[0] - 1
    out, _, _ = _make_fused(T, D, B)(logits, prefix_sum)
    return out


# speedup vs baseline: 1.2064x; 1.1387x over previous
import functools

import jax
import jax.numpy as jnp
from jax import lax
from jax.experimental import pallas as pl
from jax.experimental.pallas import tpu as pltpu
from jax.experimental.pallas import tpu_sc as plsc

NC = 2
NS = 16
LANES = 16

CH = 128
RC = 13
PC = 3


def _stage_offsets(ps_hbm, ps_v, ps_sm, B, T):
    nmem = min(LANES, B)
    pltpu.sync_copy(ps_hbm.at[pl.ds(0, nmem)], ps_v.at[pl.ds(0, nmem)])
    for k0 in range(0, B, LANES):
        v = ps_v[pl.ds(k0, LANES)]
        for k in range(min(LANES, B - k0)):
            ps_sm[k0 + k] = v[k]
    ps_sm[B] = jnp.int32(T)


def _ln(z):
    bits = lax.bitcast_convert_type(z, jnp.int32)
    eb = bits >> 23
    m = lax.bitcast_convert_type(bits - (eb << 23) + (127 << 23), jnp.float32)
    e = eb - 127
    big = m > 1.4142135623730951
    m = jnp.where(big, m * 0.5, m)
    e = e + jnp.where(big, 1, 0)
    s = (m - 1.0) / (m + 1.0)
    s2 = s * s
    p = 1.0 + s2 * (1 / 3 + s2 * (1 / 5 + s2 * (1 / 7 + s2 * (1 / 9))))
    return e.astype(jnp.float32) * 0.6931471805599453 + 2.0 * s * p


def _make_fused(T, D, B):
    COLS = D // NC
    NJ = COLS // LANES
    RW = T // NS
    assert RW == (RC + PC) * CH
    assert B == NS
    SLAB = RC * CH
    mesh = plsc.VectorSubcoreMesh(core_axis_name="c", subcore_axis_name="s")

    @functools.partial(
        pl.kernel,
        out_type=(
            jax.ShapeDtypeStruct((T, D), jnp.float32),
            jax.ShapeDtypeStruct((NC, NS, B, COLS), jnp.float32),
            jax.ShapeDtypeStruct((NC, NS, B, COLS), jnp.float32),
        ),
        compiler_params=pltpu.CompilerParams(use_tc_tiling_on_sc=False),
        mesh=mesh,
        scratch_types=[
            pltpu.VMEM((SLAB, COLS), jnp.float32),
            pltpu.VMEM((CH, COLS), jnp.float32),
            pltpu.VMEM((CH, COLS), jnp.float32),
            pltpu.VMEM((B, COLS), jnp.float32),
            pltpu.VMEM((B, COLS), jnp.float32),
            pltpu.VMEM((B, COLS), jnp.float32),
            pltpu.VMEM((1, COLS), jnp.float32),
            pltpu.VMEM((LANES,), jnp.int32),
            pltpu.SMEM((32,), jnp.int32),
            pltpu.VMEM_SHARED((B, COLS), jnp.float32),
            pltpu.SemaphoreType.DMA,
            pltpu.SemaphoreType.DMA,
            pltpu.SemaphoreType.DMA,
            pltpu.SemaphoreType.DMA,
            pltpu.SemaphoreType.DMA,
            pltpu.SemaphoreType.DMA,
        ],
    )
    def fused(x_hbm, ps_hbm, out_hbm, pm_hbm, psm_hbm, slab, bnca, bncb,
              m_v, s_v, b_v, bseg_v, ps_v, ps_sm, b_sh,
              ssem, isem_a, isem_b, oslab, osem_a, osem_b):
        cid = lax.axis_index("c")
        sid = lax.axis_index("s")
        c0 = cid * COLS
        r_lo = sid * RW

        _stage_offsets(ps_hbm, ps_v, ps_sm, B, T)

        for ci in range(RC):
            pltpu.async_copy(
                x_hbm.at[pl.ds(r_lo + ci * CH, CH), pl.ds(c0, COLS)],
                slab.at[pl.ds(ci * CH, CH), :], ssem)
        bncs = (bnca, bncb)
        isems = (isem_a, isem_b)
        osems = (osem_a, osem_b)
        for cj in range(min(PC, 2)):
            pltpu.async_copy(
                x_hbm.at[pl.ds(r_lo + (RC + cj) * CH, CH), pl.ds(c0, COLS)],
                bncs[cj], isems[cj])

        neg = jnp.full((LANES,), -jnp.inf, jnp.float32)
        zero = jnp.zeros((LANES,), jnp.float32)

        def init_body(sb, carry):
            for j in range(NJ):
                m_v[sb, pl.ds(j * LANES, LANES)] = neg
                s_v[sb, pl.ds(j * LANES, LANES)] = zero
            return carry

        lax.fori_loop(0, B, init_body, 0)

        def accum(buf, base, lo_r, hi_r, sb0):
            start = base + lo_r
            end = base + hi_r
            sb1 = jnp.int32(0)
            se = jnp.int32(0)
            for k in range(B):
                sb1 = sb1 + (ps_sm[k + 1] <= start).astype(jnp.int32)
                se = se + (ps_sm[k] < end).astype(jnp.int32)

            def seg_body(sb, carry):
                r0 = jnp.clip(ps_sm[sb] - base, lo_r, hi_r)
                r1 = jnp.clip(ps_sm[sb + 1] - base, lo_r, hi_r)
                mid = r0 + ((r1 - r0) // 4) * 4

                @plsc.parallel_loop(r0, mid, step=4, unroll=2,
                                    carry=(neg,) * NJ + (zero,) * NJ)
                def msum(r, acc):
                    x = [[buf[r + i, pl.ds(j * LANES, LANES)]
                          for j in range(NJ)] for i in range(4)]
                    out_m = []
                    out_s = []
                    for j in range(NJ):
                        bm = jnp.maximum(
                            jnp.maximum(x[0][j], x[1][j]),
                            jnp.maximum(x[2][j], x[3][j]))
                        mn = jnp.maximum(acc[j], bm)
                        sc = acc[NJ + j] * jnp.exp(acc[j] - mn)
                        e0 = jnp.exp(x[0][j] - mn) + jnp.exp(x[1][j] - mn)
                        e1 = jnp.exp(x[2][j] - mn) + jnp.exp(x[3][j] - mn)
                        out_m.append(mn)
                        out_s.append(sc + (e0 + e1))
                    return tuple(out_m) + tuple(out_s)

                @plsc.parallel_loop(mid, r1, carry=tuple(msum))
                def mtail(r, acc):
                    out_m = []
                    out_s = []
                    for j in range(NJ):
                        xr = buf[r, pl.ds(j * LANES, LANES)]
                        mn = jnp.maximum(acc[j], xr)
                        out_m.append(mn)
                        out_s.append(
                            acc[NJ + j] * jnp.exp(acc[j] - mn)
                            + jnp.exp(xr - mn))
                    return tuple(out_m) + tuple(out_s)

                for j in range(NJ):
                    sl = pl.ds(j * LANES, LANES)
                    mo = m_v[sb, sl]
                    mn = jnp.maximum(mo, mtail[j])
                    s_v[sb, sl] = (s_v[sb, sl] * jnp.exp(mo - mn)
                                   + mtail[NJ + j] * jnp.exp(mtail[j] - mn))
                    m_v[sb, sl] = mn

                return carry

            lax.fori_loop(sb1, se, seg_body, 0)
            return se - 1

        sb_t = jnp.int32(0)

        def res_body(ci, sb):
            pltpu.make_async_copy(
                x_hbm.at[pl.ds(r_lo + ci * CH, CH), pl.ds(c0, COLS)],
                slab.at[pl.ds(ci * CH, CH), :], ssem).wait()
            return accum(slab, r_lo, ci * CH, ci * CH + CH, sb)

        sb_r = lax.fori_loop(0, RC, res_body, sb_t)

        for cj in range(PC):
            base = r_lo + (RC + cj) * CH
            pltpu.make_async_copy(
                x_hbm.at[pl.ds(base, CH), pl.ds(c0, COLS)],
                bncs[cj % 2], isems[cj % 2]).wait()
            sb_r = accum(bncs[cj % 2], base, 0, CH, sb_r)
            if cj + 2 < PC:
                nbase = r_lo + (RC + cj + 2) * CH
                pltpu.async_copy(
                    x_hbm.at[pl.ds(nbase, CH), pl.ds(c0, COLS)],
                    bncs[cj % 2], isems[cj % 2])

        pltpu.sync_copy(m_v, pm_hbm.at[cid, sid])
        pltpu.sync_copy(s_v, psm_hbm.at[cid, sid])
        plsc.subcore_barrier()
        pltpu.sync_copy(pm_hbm.at[cid, :, sid, :], bnca.at[pl.ds(0, NS), :])
        pltpu.sync_copy(psm_hbm.at[cid, :, sid, :], bncb.at[pl.ds(0, NS), :])

        def mx_body(w, acc):
            return tuple(
                jnp.maximum(acc[j], bnca[w, pl.ds(j * LANES, LANES)])
                for j in range(NJ)
            )

        mseg = lax.fori_loop(0, NS, mx_body, (neg,) * NJ)

        def z_body(w, acc):
            out = []
            for j in range(NJ):
                sl = pl.ds(j * LANES, LANES)
                sp = bncb[w, sl]
                out.append(
                    acc[j]
                    + jnp.where(sp > 0, sp * jnp.exp(bnca[w, sl] - mseg[j]),
                                zero)
                )
            return tuple(out)

        zseg = lax.fori_loop(0, NS, z_body, (zero,) * NJ)

        for j in range(NJ):
            bseg_v[0, pl.ds(j * LANES, LANES)] = mseg[j] + _ln(zseg[j])
        pltpu.sync_copy(bseg_v, b_sh.at[pl.ds(sid, 1), :])
        plsc.subcore_barrier()
        pltpu.sync_copy(b_sh, b_v)

        for cj in range(min(PC, 2)):
            base = r_lo + (RC + cj) * CH
            pltpu.async_copy(
                x_hbm.at[pl.ds(base, CH), pl.ds(c0, COLS)],
                bncs[cj % 2], isems[cj % 2])

        def subtract(buf, base, lo_r, hi_r, sb0):
            start = base + lo_r
            end = base + hi_r
            sb1 = jnp.int32(0)
            se = jnp.int32(0)
            for k in range(B):
                sb1 = sb1 + (ps_sm[k + 1] <= start).astype(jnp.int32)
                se = se + (ps_sm[k] < end).astype(jnp.int32)

            def seg_body(sb, carry):
                r0 = jnp.clip(ps_sm[sb] - base, lo_r, hi_r)
                r1 = jnp.clip(ps_sm[sb + 1] - base, lo_r, hi_r)
                bj = [b_v[sb, pl.ds(j * LANES, LANES)] for j in range(NJ)]

                @plsc.parallel_loop(r0, r1, unroll=8)
                def _sub(r):
                    for j in range(NJ):
                        sl = pl.ds(j * LANES, LANES)
                        buf[r, sl] = buf[r, sl] - bj[j]

                return carry

            lax.fori_loop(sb1, se, seg_body, 0)
            return se - 1

        def out_body(ci, sb):
            sb = subtract(slab, r_lo, ci * CH, ci * CH + CH, sb)
            pltpu.async_copy(
                slab.at[pl.ds(ci * CH, CH), :],
                out_hbm.at[pl.ds(r_lo + ci * CH, CH), pl.ds(c0, COLS)], oslab)
            return sb

        sb_c = lax.fori_loop(0, RC, out_body, sb_t)

        for cj in range(PC):
            base = r_lo + (RC + cj) * CH
            pltpu.make_async_copy(
                x_hbm.at[pl.ds(base, CH), pl.ds(c0, COLS)],
                bncs[cj % 2], isems[cj % 2]).wait()
            sb_c = subtract(bncs[cj % 2], base, 0, CH, sb_c)
            pltpu.async_copy(
                bncs[cj % 2],
                out_hbm.at[pl.ds(base, CH), pl.ds(c0, COLS)], osems[cj % 2])
            if cj + 2 < PC:
                pltpu.make_async_copy(
                    bncs[cj % 2],
                    out_hbm.at[pl.ds(base, CH), pl.ds(c0, COLS)],
                    osems[cj % 2]).wait()
                pltpu.async_copy(
                    x_hbm.at[pl.ds(base + 2 * CH, CH), pl.ds(c0, COLS)],
                    bncs[cj % 2], isems[cj % 2])

        def drain_body(ci, carry):
            pltpu.make_async_copy(
                slab.at[pl.ds(ci * CH, CH), :],
                out_hbm.at[pl.ds(r_lo + ci * CH, CH), pl.ds(c0, COLS)],
                oslab).wait()
            return carry

        lax.fori_loop(0, RC, drain_body, 0)
        for cj in range(max(PC - 2, 0), PC):
            base = r_lo + (RC + cj) * CH
            pltpu.make_async_copy(
                bncs[cj % 2],
                out_hbm.at[pl.ds(base, CH), pl.ds(c0, COLS)],
                osems[cj % 2]).wait()

    return fused


def kernel(logits, prefix_sum):
    T, D = logits.shape
    B = prefix_sum.shape[0] - 1
    out, _, _ = _make_fused(T, D, B)(logits, prefix_sum)
    return out


# speedup vs baseline: 1.2224x; 1.0132x over previous
import functools

import jax
import jax.numpy as jnp
from jax import lax
from jax.experimental import pallas as pl
from jax.experimental.pallas import tpu as pltpu
from jax.experimental.pallas import tpu_sc as plsc

NC = 2
NS = 16
LANES = 16

CH = 128
RC = 13
PC = 3


def _stage_offsets(ps_hbm, ps_v, ps_sm, B, T):
    nmem = min(LANES, B)
    pltpu.sync_copy(ps_hbm.at[pl.ds(0, nmem)], ps_v.at[pl.ds(0, nmem)])
    for k0 in range(0, B, LANES):
        v = ps_v[pl.ds(k0, LANES)]
        for k in range(min(LANES, B - k0)):
            ps_sm[k0 + k] = v[k]
    ps_sm[B] = jnp.int32(T)


def _ln(z):
    bits = lax.bitcast_convert_type(z, jnp.int32)
    eb = bits >> 23
    m = lax.bitcast_convert_type(bits - (eb << 23) + (127 << 23), jnp.float32)
    e = eb - 127
    big = m > 1.4142135623730951
    m = jnp.where(big, m * 0.5, m)
    e = e + jnp.where(big, 1, 0)
    s = (m - 1.0) / (m + 1.0)
    s2 = s * s
    p = 1.0 + s2 * (1 / 3 + s2 * (1 / 5 + s2 * (1 / 7 + s2 * (1 / 9))))
    return e.astype(jnp.float32) * 0.6931471805599453 + 2.0 * s * p


def _make_fused(T, D, B):
    COLS = D // NC
    NJ = COLS // LANES
    RW = T // NS
    assert RW == (RC + PC) * CH
    assert B == NS
    SLAB = RC * CH
    mesh = plsc.VectorSubcoreMesh(core_axis_name="c", subcore_axis_name="s")

    @functools.partial(
        pl.kernel,
        out_type=(
            jax.ShapeDtypeStruct((T, D), jnp.float32),
            jax.ShapeDtypeStruct((NC, NS, B, COLS), jnp.float32),
            jax.ShapeDtypeStruct((NC, NS, B, COLS), jnp.float32),
        ),
        compiler_params=pltpu.CompilerParams(use_tc_tiling_on_sc=False),
        mesh=mesh,
        scratch_types=[
            pltpu.VMEM((SLAB, COLS), jnp.float32),
            pltpu.VMEM((CH, COLS), jnp.float32),
            pltpu.VMEM((CH, COLS), jnp.float32),
            pltpu.VMEM((B, COLS), jnp.float32),
            pltpu.VMEM((B, COLS), jnp.float32),
            pltpu.VMEM((B, COLS), jnp.float32),
            pltpu.VMEM((1, COLS), jnp.float32),
            pltpu.VMEM((LANES,), jnp.int32),
            pltpu.SMEM((32,), jnp.int32),
            pltpu.VMEM_SHARED((B, COLS), jnp.float32),
            pltpu.SemaphoreType.DMA,
            pltpu.SemaphoreType.DMA,
            pltpu.SemaphoreType.DMA,
            pltpu.SemaphoreType.DMA,
            pltpu.SemaphoreType.DMA,
            pltpu.SemaphoreType.DMA,
        ],
    )
    def fused(x_hbm, ps_hbm, out_hbm, pm_hbm, psm_hbm, slab, bnca, bncb,
              m_v, s_v, b_v, bseg_v, ps_v, ps_sm, b_sh,
              ssem, isem_a, isem_b, oslab, osem_a, osem_b):
        cid = lax.axis_index("c")
        sid = lax.axis_index("s")
        c0 = cid * COLS
        r_lo = sid * RW

        _stage_offsets(ps_hbm, ps_v, ps_sm, B, T)

        for ci in range(RC):
            pltpu.async_copy(
                x_hbm.at[pl.ds(r_lo + ci * CH, CH), pl.ds(c0, COLS)],
                slab.at[pl.ds(ci * CH, CH), :], ssem)
        bncs = (bnca, bncb)
        isems = (isem_a, isem_b)
        osems = (osem_a, osem_b)
        for cj in range(min(PC, 2)):
            pltpu.async_copy(
                x_hbm.at[pl.ds(r_lo + (RC + cj) * CH, CH), pl.ds(c0, COLS)],
                bncs[cj], isems[cj])

        neg = jnp.full((LANES,), -jnp.inf, jnp.float32)
        zero = jnp.zeros((LANES,), jnp.float32)

        def init_body(sb, carry):
            for j in range(NJ):
                m_v[sb, pl.ds(j * LANES, LANES)] = neg
                s_v[sb, pl.ds(j * LANES, LANES)] = zero
            return carry

        lax.fori_loop(0, B, init_body, 0)

        def accum(buf, base, lo_r, hi_r, sb0):
            start = base + lo_r
            end = base + hi_r
            sb1 = jnp.int32(0)
            se = jnp.int32(0)
            for k in range(B):
                sb1 = sb1 + (ps_sm[k + 1] <= start).astype(jnp.int32)
                se = se + (ps_sm[k] < end).astype(jnp.int32)

            def seg_body(sb, carry):
                r0 = jnp.clip(ps_sm[sb] - base, lo_r, hi_r)
                r1 = jnp.clip(ps_sm[sb + 1] - base, lo_r, hi_r)
                mid = r0 + ((r1 - r0) // 4) * 4

                @plsc.parallel_loop(r0, mid, step=4, unroll=2,
                                    carry=(neg,) * NJ + (zero,) * NJ)
                def msum(r, acc):
                    x = [[buf[r + i, pl.ds(j * LANES, LANES)]
                          for j in range(NJ)] for i in range(4)]
                    out_m = []
                    out_s = []
                    for j in range(NJ):
                        bm = jnp.maximum(
                            jnp.maximum(x[0][j], x[1][j]),
                            jnp.maximum(x[2][j], x[3][j]))
                        mn = jnp.maximum(acc[j], bm)
                        sc = acc[NJ + j] * jnp.exp(acc[j] - mn)
                        e0 = jnp.exp(x[0][j] - mn) + jnp.exp(x[1][j] - mn)
                        e1 = jnp.exp(x[2][j] - mn) + jnp.exp(x[3][j] - mn)
                        out_m.append(mn)
                        out_s.append(sc + (e0 + e1))
                    return tuple(out_m) + tuple(out_s)

                @plsc.parallel_loop(mid, r1, carry=tuple(msum))
                def mtail(r, acc):
                    out_m = []
                    out_s = []
                    for j in range(NJ):
                        xr = buf[r, pl.ds(j * LANES, LANES)]
                        mn = jnp.maximum(acc[j], xr)
                        out_m.append(mn)
                        out_s.append(
                            acc[NJ + j] * jnp.exp(acc[j] - mn)
                            + jnp.exp(xr - mn))
                    return tuple(out_m) + tuple(out_s)

                for j in range(NJ):
                    sl = pl.ds(j * LANES, LANES)
                    mo = m_v[sb, sl]
                    mn = jnp.maximum(mo, mtail[j])
                    s_v[sb, sl] = (s_v[sb, sl] * jnp.exp(mo - mn)
                                   + mtail[NJ + j] * jnp.exp(mtail[j] - mn))
                    m_v[sb, sl] = mn

                return carry

            lax.fori_loop(sb1, se, seg_body, 0)
            return se - 1

        sb_t = jnp.int32(0)

        def res_body(ci, sb):
            pltpu.make_async_copy(
                x_hbm.at[pl.ds(r_lo + ci * CH, CH), pl.ds(c0, COLS)],
                slab.at[pl.ds(ci * CH, CH), :], ssem).wait()
            return accum(slab, r_lo, ci * CH, ci * CH + CH, sb)

        sb_r = lax.fori_loop(0, RC, res_body, sb_t)

        for cj in range(PC):
            base = r_lo + (RC + cj) * CH
            pltpu.make_async_copy(
                x_hbm.at[pl.ds(base, CH), pl.ds(c0, COLS)],
                bncs[cj % 2], isems[cj % 2]).wait()
            sb_r = accum(bncs[cj % 2], base, 0, CH, sb_r)
            if cj + 2 < PC:
                nbase = r_lo + (RC + cj + 2) * CH
                pltpu.async_copy(
                    x_hbm.at[pl.ds(nbase, CH), pl.ds(c0, COLS)],
                    bncs[cj % 2], isems[cj % 2])

        d_pm = pltpu.async_copy(m_v, pm_hbm.at[cid, sid], osem_a)
        d_ps = pltpu.async_copy(s_v, psm_hbm.at[cid, sid], osem_b)
        d_pm.wait()
        d_ps.wait()
        plsc.subcore_barrier()
        d_pm = pltpu.async_copy(
            pm_hbm.at[cid, :, sid, :], bnca.at[pl.ds(0, NS), :], isem_a)
        d_ps = pltpu.async_copy(
            psm_hbm.at[cid, :, sid, :], bncb.at[pl.ds(0, NS), :], isem_b)
        d_pm.wait()
        d_ps.wait()

        def mx_body(w, acc):
            return tuple(
                jnp.maximum(acc[j], bnca[w, pl.ds(j * LANES, LANES)])
                for j in range(NJ)
            )

        mseg = lax.fori_loop(0, NS, mx_body, (neg,) * NJ)

        def z_body(w, acc):
            out = []
            for j in range(NJ):
                sl = pl.ds(j * LANES, LANES)
                sp = bncb[w, sl]
                out.append(
                    acc[j]
                    + jnp.where(sp > 0, sp * jnp.exp(bnca[w, sl] - mseg[j]),
                                zero)
                )
            return tuple(out)

        zseg = lax.fori_loop(0, NS, z_body, (zero,) * NJ)

        for j in range(NJ):
            bseg_v[0, pl.ds(j * LANES, LANES)] = mseg[j] + _ln(zseg[j])
        pltpu.sync_copy(bseg_v, b_sh.at[pl.ds(sid, 1), :])
        plsc.subcore_barrier()
        pltpu.sync_copy(b_sh, b_v)

        for cj in range(min(PC, 2)):
            base = r_lo + (RC + cj) * CH
            pltpu.async_copy(
                x_hbm.at[pl.ds(base, CH), pl.ds(c0, COLS)],
                bncs[cj % 2], isems[cj % 2])

        def subtract(buf, base, lo_r, hi_r, sb0):
            start = base + lo_r
            end = base + hi_r
            sb1 = jnp.int32(0)
            se = jnp.int32(0)
            for k in range(B):
                sb1 = sb1 + (ps_sm[k + 1] <= start).astype(jnp.int32)
                se = se + (ps_sm[k] < end).astype(jnp.int32)

            def seg_body(sb, carry):
                r0 = jnp.clip(ps_sm[sb] - base, lo_r, hi_r)
                r1 = jnp.clip(ps_sm[sb + 1] - base, lo_r, hi_r)
                bj = [b_v[sb, pl.ds(j * LANES, LANES)] for j in range(NJ)]

                @plsc.parallel_loop(r0, r1, unroll=8)
                def _sub(r):
                    for j in range(NJ):
                        sl = pl.ds(j * LANES, LANES)
                        buf[r, sl] = buf[r, sl] - bj[j]

                return carry

            lax.fori_loop(sb1, se, seg_body, 0)
            return se - 1

        def out_body(ci, sb):
            sb = subtract(slab, r_lo, ci * CH, ci * CH + CH, sb)
            pltpu.async_copy(
                slab.at[pl.ds(ci * CH, CH), :],
                out_hbm.at[pl.ds(r_lo + ci * CH, CH), pl.ds(c0, COLS)], oslab)
            return sb

        sb_c = lax.fori_loop(0, RC, out_body, sb_t)

        for cj in range(PC):
            base = r_lo + (RC + cj) * CH
            pltpu.make_async_copy(
                x_hbm.at[pl.ds(base, CH), pl.ds(c0, COLS)],
                bncs[cj % 2], isems[cj % 2]).wait()
            sb_c = subtract(bncs[cj % 2], base, 0, CH, sb_c)
            pltpu.async_copy(
                bncs[cj % 2],
                out_hbm.at[pl.ds(base, CH), pl.ds(c0, COLS)], osems[cj % 2])
            if cj + 2 < PC:
                pltpu.make_async_copy(
                    bncs[cj % 2],
                    out_hbm.at[pl.ds(base, CH), pl.ds(c0, COLS)],
                    osems[cj % 2]).wait()
                pltpu.async_copy(
                    x_hbm.at[pl.ds(base + 2 * CH, CH), pl.ds(c0, COLS)],
                    bncs[cj % 2], isems[cj % 2])

        def drain_body(ci, carry):
            pltpu.make_async_copy(
                slab.at[pl.ds(ci * CH, CH), :],
                out_hbm.at[pl.ds(r_lo + ci * CH, CH), pl.ds(c0, COLS)],
                oslab).wait()
            return carry

        lax.fori_loop(0, RC, drain_body, 0)
        for cj in range(max(PC - 2, 0), PC):
            base = r_lo + (RC + cj) * CH
            pltpu.make_async_copy(
                bncs[cj % 2],
                out_hbm.at[pl.ds(base, CH), pl.ds(c0, COLS)],
                osems[cj % 2]).wait()

    return fused


def kernel(logits, prefix_sum):
    T, D = logits.shape
    B = prefix_sum.shape[0] - 1
    out, _, _ = _make_fused(T, D, B)(logits, prefix_sum)
    return out
